# Initial kernel scaffold; baseline (speedup 1.0000x reference)
import jax
import jax.numpy as jnp
from jax.experimental import pallas as pl


def kernel(x, edge_index, params):
    raise NotImplementedError("write your pallas kernel here")


# speedup vs baseline: 6.9873x; 6.9873x over previous
import functools

import jax
import jax.numpy as jnp
from jax import lax
from jax.experimental import pallas as pl
from jax.experimental.pallas import tpu as pltpu
from jax.experimental.pallas import tpu_sc as plsc

NC = 2
NS = 16
K = 128


def _make_agg_kernel(n_rows, dh, j_chunks):
    mesh = plsc.VectorSubcoreMesh(
        core_axis_name="c", subcore_axis_name="s", num_cores=NC,
        num_subcores=NS)
    rows_per_tile = n_rows // NS
    init_chunks = rows_per_tile // K

    @functools.partial(
        pl.kernel,
        out_type=jax.ShapeDtypeStruct((NC, n_rows, dh), jnp.float32),
        mesh=mesh,
        scratch_types=[
            pltpu.VMEM((j_chunks, K), jnp.int32),
            pltpu.VMEM((j_chunks, K), jnp.int32),
            pltpu.VMEM((K, dh), jnp.float32),
            pltpu.VMEM((K, dh), jnp.float32),
            pltpu.VMEM_SHARED((n_rows, dh), jnp.float32),
            pltpu.SemaphoreType.DMA,
            pltpu.SemaphoreType.DMA,
        ],
        compiler_params=pltpu.CompilerParams(use_tc_tiling_on_sc=False),
    )
    def agg(h_hbm, src_hbm, dst_hbm, out_hbm, src_v, dst_v, rows0, rows1,
            acc, sem0, sem1):
        cid = lax.axis_index("c")
        sid = lax.axis_index("s")
        pltpu.sync_copy(src_hbm.at[sid], src_v)
        pltpu.sync_copy(dst_hbm.at[sid], dst_v)
        base = sid * rows_per_tile
        for i in range(init_chunks):
            pltpu.sync_copy(h_hbm.at[cid, pl.ds(base + i * K, K)], rows0)
            pltpu.sync_copy(rows0, acc.at[pl.ds(base + i * K, K)])
        plsc.subcore_barrier()

        @pl.loop(0, j_chunks // 2)
        def _(g):
            j0 = g * 2
            j1 = j0 + 1
            c0 = pltpu.async_copy(h_hbm.at[cid].at[src_v.at[j0]], rows0, sem0)
            c1 = pltpu.async_copy(h_hbm.at[cid].at[src_v.at[j1]], rows1, sem1)
            c0.wait()
            pltpu.sync_copy(rows0, acc.at[dst_v.at[j0]], add=True)
            c1.wait()
            pltpu.sync_copy(rows1, acc.at[dst_v.at[j1]], add=True)

        plsc.subcore_barrier()
        for i in range(init_chunks):
            pltpu.sync_copy(acc.at[pl.ds(base + i * K, K)], rows0)
            pltpu.sync_copy(rows0, out_hbm.at[cid, pl.ds(base + i * K, K)])

    return agg


def _mlp_call(a, Ws, bs, n_rows, blk, split_out):
    dh = a.shape[-1]
    d_out = Ws[2].shape[1]

    def body(a_r, w0, b0, w1, b1, w2, b2, o_r):
        h = jnp.concatenate([a_r[0], a_r[1]], axis=1)
        h = jnp.maximum(
            jnp.dot(h, w0[...], preferred_element_type=jnp.float32) + b0[...],
            0.0)
        h = jnp.maximum(
            jnp.dot(h, w1[...], preferred_element_type=jnp.float32) + b1[...],
            0.0)
        h = jnp.dot(h, w2[...], preferred_element_type=jnp.float32) + b2[...]
        if split_out:
            o_r[0] = h[:, :dh]
            o_r[1] = h[:, dh:]
        else:
            o_r[...] = h

    full = lambda w: pl.BlockSpec(w.shape, lambda i: (0, 0))
    if split_out:
        out_spec = pl.BlockSpec((NC, blk, dh), lambda i: (0, i, 0))
        out_shape = jax.ShapeDtypeStruct((NC, n_rows, dh), jnp.float32)
    else:
        out_spec = pl.BlockSpec((blk, d_out), lambda i: (i, 0))
        out_shape = jax.ShapeDtypeStruct((n_rows, d_out), jnp.float32)
    return pl.pallas_call(
        body,
        grid=(n_rows // blk,),
        in_specs=[
            pl.BlockSpec((NC, blk, dh), lambda i: (0, i, 0)),
            full(Ws[0]), full(bs[0]), full(Ws[1]), full(bs[1]),
            full(Ws[2]), full(bs[2]),
        ],
        out_specs=out_spec,
        out_shape=out_shape,
    )(a, Ws[0], bs[0], Ws[1], bs[1], Ws[2], bs[2])


def kernel(x, edge_index, params):
    n, d = x.shape
    dh = d // NC
    e = edge_index.shape[1]
    n_rows = ((n + NS * K - 1) // (NS * K)) * NS * K
    slots_pt = ((e + NS * K - 1) // (NS * K) + 1) // 2 * 2 * K
    j_chunks = slots_pt // K
    pad = NS * slots_pt - e

    src = edge_index[0].astype(jnp.int32)
    dst = edge_index[1].astype(jnp.int32)
    pad_src = jnp.arange(pad, dtype=jnp.int32) % n
    pad_dst = n + jnp.arange(pad, dtype=jnp.int32) % (n_rows - n)
    src_w = jnp.concatenate([src, pad_src]).reshape(NS, j_chunks, K)
    dst_w = jnp.concatenate([dst, pad_dst]).reshape(NS, j_chunks, K)

    x_pad = jnp.zeros((n_rows, d), jnp.float32).at[:n].set(x)
    h = x_pad.reshape(n_rows, NC, dh).transpose(1, 0, 2)

    agg = _make_agg_kernel(n_rows, dh, j_chunks)

    for li, (Ws, bs) in enumerate(params):
        a = agg(h, src_w, dst_w)
        bs2 = tuple(b.reshape(1, -1) for b in bs)
        h = _mlp_call(a, Ws, bs2, n_rows, 1024, split_out=(li < len(params) - 1))
    return h[:n]


# speedup vs baseline: 10.9492x; 1.5670x over previous
import functools

import jax
import jax.numpy as jnp
from jax import lax
from jax.experimental import pallas as pl
from jax.experimental.pallas import tpu as pltpu
from jax.experimental.pallas import tpu_sc as plsc

NC = 2
NS = 16
K = 128
NBUF = 4


def _make_agg_kernel(n_rows, dh, j_chunks):
    mesh = plsc.VectorSubcoreMesh(
        core_axis_name="c", subcore_axis_name="s", num_cores=NC,
        num_subcores=NS)
    rows_per_tile = n_rows // NS
    init_chunks = rows_per_tile // K

    @functools.partial(
        pl.kernel,
        out_type=jax.ShapeDtypeStruct((NC, n_rows, dh), jnp.float32),
        mesh=mesh,
        scratch_types=[
            pltpu.VMEM((j_chunks, K), jnp.int32),
            pltpu.VMEM((j_chunks, K), jnp.int32),
            [pltpu.VMEM((K, dh), jnp.float32) for _ in range(NBUF)],
            pltpu.VMEM_SHARED((n_rows, dh), jnp.float32),
            [pltpu.SemaphoreType.DMA for _ in range(NBUF + 2)],
        ],
        compiler_params=pltpu.CompilerParams(use_tc_tiling_on_sc=False),
    )
    def agg(h_hbm, src_hbm, dst_hbm, out_hbm, src_v, dst_v, rows, acc, sems):
        cid = lax.axis_index("c")
        sid = lax.axis_index("s")
        ci = pltpu.async_copy(src_hbm.at[sid], src_v, sems[NBUF])
        cj = pltpu.async_copy(dst_hbm.at[sid], dst_v, sems[NBUF + 1])
        base = sid * rows_per_tile
        for i in range(init_chunks):
            pltpu.sync_copy(h_hbm.at[cid, pl.ds(base + i * K, K)],
                            acc.at[pl.ds(base + i * K, K)])
        ci.wait()
        cj.wait()
        plsc.subcore_barrier()

        def gather(j, b):
            return pltpu.async_copy(h_hbm.at[cid].at[src_v.at[j]], rows[b],
                                    sems[b])

        def drain_and_scatter(j, b):
            pltpu.make_async_copy(h_hbm.at[cid].at[src_v.at[j]], rows[b],
                                  sems[b]).wait()
            pltpu.sync_copy(rows[b], acc.at[dst_v.at[j]], add=True)

        for b in range(NBUF):
            gather(b, b)

        @pl.loop(0, j_chunks // NBUF - 1)
        def _(g):
            for b in range(NBUF):
                j = g * NBUF + b
                drain_and_scatter(j, b)
                gather(j + NBUF, b)

        for b in range(NBUF):
            drain_and_scatter(j_chunks - NBUF + b, b)

        plsc.subcore_barrier()
        for i in range(init_chunks):
            pltpu.sync_copy(acc.at[pl.ds(base + i * K, K)],
                            out_hbm.at[cid, pl.ds(base + i * K, K)])

    return agg


def _mlp_call(a, Ws, bs, n_rows, blk, split_out):
    dh = a.shape[-1]
    d_out = Ws[2].shape[1]

    def body(a_r, w0, b0, w1, b1, w2, b2, o_r):
        h = jnp.concatenate([a_r[0], a_r[1]], axis=1)
        h = jnp.maximum(
            jnp.dot(h, w0[...], preferred_element_type=jnp.float32) + b0[...],
            0.0)
        h = jnp.maximum(
            jnp.dot(h, w1[...], preferred_element_type=jnp.float32) + b1[...],
            0.0)
        h = jnp.dot(h, w2[...], preferred_element_type=jnp.float32) + b2[...]
        if split_out:
            o_r[0] = h[:, :dh]
            o_r[1] = h[:, dh:]
        else:
            o_r[...] = h

    full = lambda w: pl.BlockSpec(w.shape, lambda i: (0, 0))
    if split_out:
        out_spec = pl.BlockSpec((NC, blk, dh), lambda i: (0, i, 0))
        out_shape = jax.ShapeDtypeStruct((NC, n_rows, dh), jnp.float32)
    else:
        out_spec = pl.BlockSpec((blk, d_out), lambda i: (i, 0))
        out_shape = jax.ShapeDtypeStruct((n_rows, d_out), jnp.float32)
    return pl.pallas_call(
        body,
        grid=(n_rows // blk,),
        in_specs=[
            pl.BlockSpec((NC, blk, dh), lambda i: (0, i, 0)),
            full(Ws[0]), full(bs[0]), full(Ws[1]), full(bs[1]),
            full(Ws[2]), full(bs[2]),
        ],
        out_specs=out_spec,
        out_shape=out_shape,
    )(a, Ws[0], bs[0], Ws[1], bs[1], Ws[2], bs[2])


def kernel(x, edge_index, params):
    n, d = x.shape
    dh = d // NC
    e = edge_index.shape[1]
    n_rows = ((n + NS * K - 1) // (NS * K)) * NS * K
    j_chunks = -(-(-(-e // (NS * K))) // NBUF) * NBUF
    j_chunks = max(j_chunks, 2 * NBUF)
    slots_pt = j_chunks * K
    pad = NS * slots_pt - e

    src = edge_index[0].astype(jnp.int32)
    dst = edge_index[1].astype(jnp.int32)
    pad_src = jnp.arange(pad, dtype=jnp.int32) % n
    pad_dst = n + jnp.arange(pad, dtype=jnp.int32) % (n_rows - n)
    src_w = jnp.concatenate([src, pad_src]).reshape(NS, j_chunks, K)
    dst_w = jnp.concatenate([dst, pad_dst]).reshape(NS, j_chunks, K)

    x_pad = jnp.zeros((n_rows, d), jnp.float32).at[:n].set(x)
    h = x_pad.reshape(n_rows, NC, dh).transpose(1, 0, 2)

    agg = _make_agg_kernel(n_rows, dh, j_chunks)

    for li, (Ws, bs) in enumerate(params):
        a = agg(h, src_w, dst_w)
        bs2 = tuple(b.reshape(1, -1) for b in bs)
        h = _mlp_call(a, Ws, bs2, n_rows, 1024, split_out=(li < len(params) - 1))
    return h[:n]
